# Initial kernel scaffold; baseline (speedup 1.0000x reference)
import jax
import jax.numpy as jnp
from jax.experimental import pallas as pl


def kernel(types, system_ids, weights, type_to_index):
    raise NotImplementedError("write your pallas kernel here")


# speedup vs baseline: 203.4426x; 203.4426x over previous
import functools

import jax
import jax.numpy as jnp
from jax import lax
from jax.experimental import pallas as pl
from jax.experimental.pallas import tpu as pltpu
from jax.experimental.pallas import tpu_sc as plsc

N_ATOMS = 1048576
N_SYSTEMS = 4096
N_TYPES = 100
N_PROPS = 32

NUM_CORES = 2
NUM_SUBCORES = 16
NUM_WORKERS = NUM_CORES * NUM_SUBCORES
SYS_PER_W = N_SYSTEMS // NUM_WORKERS
TPAD = 128
HIST_WORDS = SYS_PER_W * TPAD
CHUNK = 4096
SEARCH_ITERS = 21


def _sc_hist_body(types_hbm, sys_hbm, t2i_hbm, out_hbm,
                  t2i_v, tbuf, sbuf, hist, probe1, probe2, sem1, sem2):
    wid = lax.axis_index("c") * NUM_SUBCORES + lax.axis_index("s")
    lo_sys = wid * SYS_PER_W
    hi_sys = lo_sys + SYS_PER_W

    zeros16 = jnp.zeros((16,), jnp.float32)
    ones16 = jnp.ones((16,), jnp.float32)
    lanes = lax.broadcasted_iota(jnp.int32, (16,), 0)

    def _zero(i, carry):
        hist[pl.ds(i * 16, 16)] = zeros16
        return carry
    lax.fori_loop(0, HIST_WORDS // 16, _zero, 0)

    pltpu.sync_copy(t2i_hbm, t2i_v)

    def _probe_val(pbuf, pm):
        v = pbuf[...]
        return jnp.max(jnp.where(lanes == (pm & 15), v, 0))

    def _sbody(_, carry):
        lo1, hi1, lo2, hi2 = carry
        mid1 = (lo1 + hi1) >> 1
        mid2 = (lo2 + hi2) >> 1
        pm1 = jnp.minimum(mid1, N_ATOMS - 1)
        pm2 = jnp.minimum(mid2, N_ATOMS - 1)
        b1 = pl.multiple_of(pm1 & -16, 16)
        b2 = pl.multiple_of(pm2 & -16, 16)
        d1 = pltpu.async_copy(sys_hbm.at[pl.ds(b1, 16)], probe1, sem1)
        d2 = pltpu.async_copy(sys_hbm.at[pl.ds(b2, 16)], probe2, sem2)
        d1.wait()
        d2.wait()
        x1 = _probe_val(probe1, pm1)
        x2 = _probe_val(probe2, pm2)
        act1 = lo1 < hi1
        act2 = lo2 < hi2
        p1 = x1 < lo_sys
        p2 = x2 < hi_sys
        lo1n = jnp.where(act1, jnp.where(p1, mid1 + 1, lo1), lo1)
        hi1n = jnp.where(act1, jnp.where(p1, hi1, mid1), hi1)
        lo2n = jnp.where(act2, jnp.where(p2, mid2 + 1, lo2), lo2)
        hi2n = jnp.where(act2, jnp.where(p2, hi2, mid2), hi2)
        return (lo1n, hi1n, lo2n, hi2n)

    z = jnp.int32(0)
    n = jnp.int32(N_ATOMS)
    start, _, end, _ = lax.fori_loop(0, SEARCH_ITERS, _sbody, (z, n, z, n))

    c0 = start // CHUNK
    c1 = (end + CHUNK - 1) // CHUNK

    def _chunk(c, carry):
        off = pl.multiple_of(c * CHUNK, CHUNK)
        da = pltpu.async_copy(types_hbm.at[pl.ds(off, CHUNK)], tbuf, sem1)
        db = pltpu.async_copy(sys_hbm.at[pl.ds(off, CHUNK)], sbuf, sem2)
        da.wait()
        db.wait()

        def _vec(i, carry2):
            t = tbuf[pl.ds(i * 16, 16)]
            s = sbuf[pl.ds(i * 16, 16)]
            row = plsc.load_gather(t2i_v, [t])
            key = (s - lo_sys) * TPAD + row
            key = jnp.clip(key, 0, HIST_WORDS - 1)
            m = (s >= lo_sys) & (s < hi_sys)
            plsc.addupdate_scatter(hist, [key], ones16, mask=m)
            return carry2
        lax.fori_loop(0, CHUNK // 16, _vec, 0)
        return carry

    lax.fori_loop(c0, c1, _chunk, 0)

    dst = pl.multiple_of(wid * HIST_WORDS, HIST_WORDS)
    pltpu.sync_copy(hist, out_hbm.at[pl.ds(dst, HIST_WORDS)])


@functools.partial(jax.jit, static_argnames=())
def _sc_hist(types_i, sys_i, t2i_pad):
    mesh = plsc.VectorSubcoreMesh(
        core_axis_name="c", subcore_axis_name="s",
        num_cores=NUM_CORES, num_subcores=NUM_SUBCORES)
    f = pl.kernel(
        _sc_hist_body,
        out_type=jax.ShapeDtypeStruct((N_SYSTEMS * TPAD,), jnp.float32),
        mesh=mesh,
        scratch_types=[
            pltpu.VMEM((TPAD,), jnp.int32),
            pltpu.VMEM((CHUNK,), jnp.int32),
            pltpu.VMEM((CHUNK,), jnp.int32),
            pltpu.VMEM((HIST_WORDS,), jnp.float32),
            pltpu.VMEM((16,), jnp.int32),
            pltpu.VMEM((16,), jnp.int32),
            pltpu.SemaphoreType.DMA,
            pltpu.SemaphoreType.DMA,
        ],
        compiler_params=pltpu.CompilerParams(needs_layout_passes=False),
    )
    return f(types_i, sys_i, t2i_pad)


def _mm_body(c_ref, w_ref, o_ref):
    o_ref[...] = jnp.dot(c_ref[...], w_ref[...],
                         preferred_element_type=jnp.float32)


def kernel(types, system_ids, weights, type_to_index):
    types_i = types.astype(jnp.int32)
    sys_i = system_ids.astype(jnp.int32)
    t2i_pad = jnp.zeros((TPAD,), jnp.int32).at[:N_TYPES].set(
        type_to_index.astype(jnp.int32))
    w_pad = jnp.zeros((TPAD, N_PROPS), jnp.float32).at[:N_TYPES].set(
        weights.astype(jnp.float32))

    counts = _sc_hist(types_i, sys_i, t2i_pad).reshape(N_SYSTEMS, TPAD)

    out = pl.pallas_call(
        _mm_body,
        out_shape=jax.ShapeDtypeStruct((N_SYSTEMS, N_PROPS), jnp.float32),
    )(counts, w_pad)
    return out


# speedup vs baseline: 310.4520x; 1.5260x over previous
import functools

import jax
import jax.numpy as jnp
from jax import lax
from jax.experimental import pallas as pl
from jax.experimental.pallas import tpu as pltpu
from jax.experimental.pallas import tpu_sc as plsc

N_ATOMS = 1048576
N_SYSTEMS = 4096
N_TYPES = 100
N_PROPS = 32

NUM_CORES = 2
NUM_SUBCORES = 16
NUM_WORKERS = NUM_CORES * NUM_SUBCORES
SYS_PER_W = N_SYSTEMS // NUM_WORKERS
TPAD = 128
HIST_WORDS = SYS_PER_W * TPAD
CHUNK = 4096
SEARCH_ITERS = 21


def _sc_hist_body(types_hbm, sys_hbm, t2i_hbm, out_hbm,
                  t2i_v, tbuf, sbuf, hist, probe1, probe2, sem1, sem2):
    wid = lax.axis_index("c") * NUM_SUBCORES + lax.axis_index("s")
    lo_sys = wid * SYS_PER_W
    hi_sys = lo_sys + SYS_PER_W

    zeros16 = jnp.zeros((16,), jnp.float32)
    ones16 = jnp.ones((16,), jnp.float32)
    lanes = lax.broadcasted_iota(jnp.int32, (16,), 0)

    def _zero(i, carry):
        hist[pl.ds(i * 16, 16)] = zeros16
        return carry
    lax.fori_loop(0, HIST_WORDS // 16, _zero, 0)

    pltpu.sync_copy(t2i_hbm, t2i_v)

    def _probe_val(pbuf, pm):
        v = pbuf[...]
        return jnp.max(jnp.where(lanes == (pm & 15), v, 0))

    def _sbody(_, carry):
        lo1, hi1, lo2, hi2 = carry
        mid1 = (lo1 + hi1) >> 1
        mid2 = (lo2 + hi2) >> 1
        pm1 = jnp.minimum(mid1, N_ATOMS - 1)
        pm2 = jnp.minimum(mid2, N_ATOMS - 1)
        b1 = pl.multiple_of(pm1 & -16, 16)
        b2 = pl.multiple_of(pm2 & -16, 16)
        d1 = pltpu.async_copy(sys_hbm.at[pl.ds(b1, 16)], probe1, sem1)
        d2 = pltpu.async_copy(sys_hbm.at[pl.ds(b2, 16)], probe2, sem2)
        d1.wait()
        d2.wait()
        x1 = _probe_val(probe1, pm1)
        x2 = _probe_val(probe2, pm2)
        act1 = lo1 < hi1
        act2 = lo2 < hi2
        p1 = x1 < lo_sys
        p2 = x2 < hi_sys
        lo1n = jnp.where(act1, jnp.where(p1, mid1 + 1, lo1), lo1)
        hi1n = jnp.where(act1, jnp.where(p1, hi1, mid1), hi1)
        lo2n = jnp.where(act2, jnp.where(p2, mid2 + 1, lo2), lo2)
        hi2n = jnp.where(act2, jnp.where(p2, hi2, mid2), hi2)
        return (lo1n, hi1n, lo2n, hi2n)

    z = jnp.int32(0)
    n = jnp.int32(N_ATOMS)
    start, _, end, _ = lax.fori_loop(0, SEARCH_ITERS, _sbody, (z, n, z, n))

    c0 = start // CHUNK
    c1 = (end + CHUNK - 1) // CHUNK
    ci0 = jnp.minimum(jnp.maximum((start + CHUNK - 1) // CHUNK, c0), c1)
    ci1 = jnp.minimum(jnp.maximum(end // CHUNK, ci0), c1)

    def _load_chunk(c):
        off = pl.multiple_of(c * CHUNK, CHUNK)
        da = pltpu.async_copy(types_hbm.at[pl.ds(off, CHUNK)], tbuf, sem1)
        db = pltpu.async_copy(sys_hbm.at[pl.ds(off, CHUNK)], sbuf, sem2)
        da.wait()
        db.wait()

    def _chunk_masked(c, carry):
        _load_chunk(c)

        @plsc.parallel_loop(0, CHUNK, 16, unroll=4)
        def _vec(i):
            t = tbuf[pl.ds(i, 16)]
            s = sbuf[pl.ds(i, 16)]
            row = plsc.load_gather(t2i_v, [t])
            key = ((s - lo_sys) << 7) + row
            key = jnp.clip(key, 0, HIST_WORDS - 1)
            m = (s >= lo_sys) & (s < hi_sys)
            plsc.addupdate_scatter(hist, [key], ones16, mask=m)
        return carry

    def _chunk_inner(c, carry):
        _load_chunk(c)

        @plsc.parallel_loop(0, CHUNK, 16, unroll=8)
        def _vec(i):
            t = tbuf[pl.ds(i, 16)]
            s = sbuf[pl.ds(i, 16)]
            row = plsc.load_gather(t2i_v, [t])
            key = ((s - lo_sys) << 7) + row
            plsc.addupdate_scatter(hist, [key], ones16)
        return carry

    lax.fori_loop(c0, ci0, _chunk_masked, 0)
    lax.fori_loop(ci0, ci1, _chunk_inner, 0)
    lax.fori_loop(ci1, c1, _chunk_masked, 0)

    dst = pl.multiple_of(wid * HIST_WORDS, HIST_WORDS)
    pltpu.sync_copy(hist, out_hbm.at[pl.ds(dst, HIST_WORDS)])


@functools.partial(jax.jit, static_argnames=())
def _sc_hist(types_i, sys_i, t2i_pad):
    mesh = plsc.VectorSubcoreMesh(
        core_axis_name="c", subcore_axis_name="s",
        num_cores=NUM_CORES, num_subcores=NUM_SUBCORES)
    f = pl.kernel(
        _sc_hist_body,
        out_type=jax.ShapeDtypeStruct((N_SYSTEMS * TPAD,), jnp.float32),
        mesh=mesh,
        scratch_types=[
            pltpu.VMEM((TPAD,), jnp.int32),
            pltpu.VMEM((CHUNK,), jnp.int32),
            pltpu.VMEM((CHUNK,), jnp.int32),
            pltpu.VMEM((HIST_WORDS,), jnp.float32),
            pltpu.VMEM((16,), jnp.int32),
            pltpu.VMEM((16,), jnp.int32),
            pltpu.SemaphoreType.DMA,
            pltpu.SemaphoreType.DMA,
        ],
        compiler_params=pltpu.CompilerParams(needs_layout_passes=False),
    )
    return f(types_i, sys_i, t2i_pad)


def _mm_body(c_ref, w_ref, o_ref):
    o_ref[...] = jnp.dot(c_ref[...], w_ref[...],
                         preferred_element_type=jnp.float32)


def kernel(types, system_ids, weights, type_to_index):
    types_i = types.astype(jnp.int32)
    sys_i = system_ids.astype(jnp.int32)
    t2i_pad = jnp.zeros((TPAD,), jnp.int32).at[:N_TYPES].set(
        type_to_index.astype(jnp.int32))
    w_pad = jnp.zeros((TPAD, N_PROPS), jnp.float32).at[:N_TYPES].set(
        weights.astype(jnp.float32))

    counts = _sc_hist(types_i, sys_i, t2i_pad).reshape(N_SYSTEMS, TPAD)

    out = pl.pallas_call(
        _mm_body,
        out_shape=jax.ShapeDtypeStruct((N_SYSTEMS, N_PROPS), jnp.float32),
    )(counts, w_pad)
    return out


# speedup vs baseline: 349.3506x; 1.1253x over previous
import functools

import jax
import jax.numpy as jnp
from jax import lax
from jax.experimental import pallas as pl
from jax.experimental.pallas import tpu as pltpu
from jax.experimental.pallas import tpu_sc as plsc

N_ATOMS = 1048576
N_SYSTEMS = 4096
N_TYPES = 100
N_PROPS = 32

NUM_CORES = 2
NUM_SUBCORES = 16
NUM_WORKERS = NUM_CORES * NUM_SUBCORES
SYS_PER_W = N_SYSTEMS // NUM_WORKERS
TPAD = 128
HIST_WORDS = SYS_PER_W * TPAD
CHUNK = 8192
SEARCH_ITERS = 5


def _sc_hist_body(types_hbm, sys_hbm, t2i_hbm, out_hbm,
                  t2i_v, tbuf, sbuf, hist, probe1, probe2, sem1, sem2):
    wid = lax.axis_index("c") * NUM_SUBCORES + lax.axis_index("s")
    lo_sys = wid * SYS_PER_W
    hi_sys = lo_sys + SYS_PER_W

    zeros16 = jnp.zeros((16,), jnp.float32)
    ones16 = jnp.ones((16,), jnp.float32)
    lanes = lax.broadcasted_iota(jnp.int32, (16,), 0)

    def _zero(i, carry):
        hist[pl.ds(i * 16, 16)] = zeros16
        return carry
    lax.fori_loop(0, HIST_WORDS // 16, _zero, 0)

    pltpu.sync_copy(t2i_hbm, t2i_v)

    def _round(lo, hi, target, pbuf, sem):
        s = (hi - lo + 15) >> 4
        p = lo + lanes * s
        pc = jnp.minimum(p, N_ATOMS - 1)
        d = pltpu.async_copy(sys_hbm.at[pc], pbuf, sem)
        return p, s, d

    def _update(lo, hi, target, p, s, pbuf):
        vals = pbuf[...]
        pred = (vals < target) & (p < hi)
        k = jnp.max(plsc.all_reduce_population_count(pred))
        lo_n = jnp.where(k > 0, lo + (k - 1) * s + 1, lo)
        hi_n = jnp.where(k < 16, jnp.minimum(hi, lo + k * s), hi)
        return lo_n, jnp.maximum(hi_n, lo_n)

    def _sbody(_, carry):
        lo1, hi1, lo2, hi2 = carry
        p1, s1, d1 = _round(lo1, hi1, lo_sys, probe1, sem1)
        p2, s2, d2 = _round(lo2, hi2, hi_sys, probe2, sem2)
        d1.wait()
        d2.wait()
        lo1n, hi1n = _update(lo1, hi1, lo_sys, p1, s1, probe1)
        lo2n, hi2n = _update(lo2, hi2, hi_sys, p2, s2, probe2)
        return (lo1n, hi1n, lo2n, hi2n)

    z = jnp.int32(0)
    n = jnp.int32(N_ATOMS)
    start, _, end, _ = lax.fori_loop(0, SEARCH_ITERS, _sbody, (z, n, z, n))

    c0 = start // CHUNK
    c1 = (end + CHUNK - 1) // CHUNK
    ci0 = jnp.minimum(jnp.maximum((start + CHUNK - 1) // CHUNK, c0), c1)
    ci1 = jnp.minimum(jnp.maximum(end // CHUNK, ci0), c1)

    def _load_chunk(c):
        off = pl.multiple_of(c * CHUNK, CHUNK)
        da = pltpu.async_copy(types_hbm.at[pl.ds(off, CHUNK)], tbuf, sem1)
        db = pltpu.async_copy(sys_hbm.at[pl.ds(off, CHUNK)], sbuf, sem2)
        da.wait()
        db.wait()

    def _chunk_masked(c, carry):
        _load_chunk(c)

        @plsc.parallel_loop(0, CHUNK, 16, unroll=4)
        def _vec(i):
            t = tbuf[pl.ds(i, 16)]
            s = sbuf[pl.ds(i, 16)]
            row = plsc.load_gather(t2i_v, [t])
            key = ((s - lo_sys) << 7) + row
            key = jnp.clip(key, 0, HIST_WORDS - 1)
            m = (s >= lo_sys) & (s < hi_sys)
            plsc.addupdate_scatter(hist, [key], ones16, mask=m)
        return carry

    def _chunk_inner(c, carry):
        _load_chunk(c)

        @plsc.parallel_loop(0, CHUNK, 16, unroll=8)
        def _vec(i):
            t = tbuf[pl.ds(i, 16)]
            s = sbuf[pl.ds(i, 16)]
            row = plsc.load_gather(t2i_v, [t])
            key = ((s - lo_sys) << 7) + row
            plsc.addupdate_scatter(hist, [key], ones16)
        return carry

    lax.fori_loop(c0, ci0, _chunk_masked, 0)
    lax.fori_loop(ci0, ci1, _chunk_inner, 0)
    lax.fori_loop(ci1, c1, _chunk_masked, 0)

    dst = pl.multiple_of(wid * HIST_WORDS, HIST_WORDS)
    pltpu.sync_copy(hist, out_hbm.at[pl.ds(dst, HIST_WORDS)])


@functools.partial(jax.jit, static_argnames=())
def _sc_hist(types_i, sys_i, t2i_pad):
    mesh = plsc.VectorSubcoreMesh(
        core_axis_name="c", subcore_axis_name="s",
        num_cores=NUM_CORES, num_subcores=NUM_SUBCORES)
    f = pl.kernel(
        _sc_hist_body,
        out_type=jax.ShapeDtypeStruct((N_SYSTEMS * TPAD,), jnp.float32),
        mesh=mesh,
        scratch_types=[
            pltpu.VMEM((TPAD,), jnp.int32),
            pltpu.VMEM((CHUNK,), jnp.int32),
            pltpu.VMEM((CHUNK,), jnp.int32),
            pltpu.VMEM((HIST_WORDS,), jnp.float32),
            pltpu.VMEM((16,), jnp.int32),
            pltpu.VMEM((16,), jnp.int32),
            pltpu.SemaphoreType.DMA,
            pltpu.SemaphoreType.DMA,
        ],
        compiler_params=pltpu.CompilerParams(needs_layout_passes=False),
    )
    return f(types_i, sys_i, t2i_pad)


def _mm_body(c_ref, w_ref, o_ref):
    o_ref[...] = jnp.dot(c_ref[...], w_ref[...],
                         preferred_element_type=jnp.float32)


def kernel(types, system_ids, weights, type_to_index):
    types_i = types.astype(jnp.int32)
    sys_i = system_ids.astype(jnp.int32)
    t2i_pad = jnp.zeros((TPAD,), jnp.int32).at[:N_TYPES].set(
        type_to_index.astype(jnp.int32))
    w_pad = jnp.zeros((TPAD, N_PROPS), jnp.float32).at[:N_TYPES].set(
        weights.astype(jnp.float32))

    counts = _sc_hist(types_i, sys_i, t2i_pad).reshape(N_SYSTEMS, TPAD)

    out = pl.pallas_call(
        _mm_body,
        out_shape=jax.ShapeDtypeStruct((N_SYSTEMS, N_PROPS), jnp.float32),
    )(counts, w_pad)
    return out
